# Initial kernel scaffold; baseline (speedup 1.0000x reference)
import jax
import jax.numpy as jnp
from jax.experimental import pallas as pl


def kernel(edge_index, x, W, b):
    raise NotImplementedError("write your pallas kernel here")


# speedup vs baseline: 10.0845x; 10.0845x over previous
import functools

import jax
import jax.numpy as jnp
from jax import lax
from jax.experimental import pallas as pl
from jax.experimental.pallas import tpu as pltpu
from jax.experimental.pallas import tpu_sc as plsc

N = 10000
N_PAD = 10240
E = 320000
D = 128
C = 128
NC = 2
NS = 16
NW = NC * NS
ITERS = -(-E // (NW * C))
E_PER_W = ITERS * C
E_PAD = NW * E_PER_W
RPS = N_PAD // NS

_mesh = plsc.VectorSubcoreMesh(core_axis_name="c", subcore_axis_name="s")
_f32 = jnp.float32


def _wid():
    return lax.axis_index("s") * NC + lax.axis_index("c")


@functools.partial(
    pl.kernel,
    out_type=jax.ShapeDtypeStruct((NC, N_PAD, D), _f32),
    mesh=_mesh,
    scratch_types=[
        pltpu.VMEM((C,), jnp.int32),
        pltpu.VMEM((C, D), _f32),
        pltpu.VMEM_SHARED((N_PAD, D), _f32),
    ],
)
def _deg_kernel(dst_hbm, zeros_hbm, ones_hbm, out_hbm, dst_v, ones_v, acc):
    cid = lax.axis_index("c")
    sid = lax.axis_index("s")
    wid = _wid()

    pltpu.sync_copy(ones_hbm, ones_v)

    rbase = sid * RPS
    pltpu.sync_copy(zeros_hbm.at[pl.ds(rbase, RPS)], acc.at[pl.ds(rbase, RPS)])
    plsc.subcore_barrier()

    ebase = wid * E_PER_W

    @pl.loop(0, ITERS)
    def _(i):
        pltpu.sync_copy(dst_hbm.at[pl.ds(ebase + i * C, C)], dst_v)
        pltpu.sync_copy(ones_v, acc.at[dst_v], add=True)

    plsc.subcore_barrier()
    pltpu.sync_copy(acc.at[pl.ds(rbase, RPS)], out_hbm.at[cid, pl.ds(rbase, RPS)])


@functools.partial(
    pl.kernel,
    out_type=jax.ShapeDtypeStruct((NC, N_PAD, D), _f32),
    mesh=_mesh,
    scratch_types=[
        pltpu.VMEM((C,), jnp.int32),
        pltpu.VMEM((C,), jnp.int32),
        pltpu.VMEM((C, D), _f32),
        pltpu.VMEM_SHARED((N_PAD, D), _f32),
        pltpu.SemaphoreType.DMA,
    ],
)
def _hop_kernel(y_hbm, src_hbm, dst_hbm, zeros_hbm, out_hbm,
                src_v, dst_v, rows_v, acc, sem):
    cid = lax.axis_index("c")
    sid = lax.axis_index("s")
    wid = _wid()

    rbase = sid * RPS
    pltpu.sync_copy(zeros_hbm.at[pl.ds(rbase, RPS)], acc.at[pl.ds(rbase, RPS)])
    plsc.subcore_barrier()

    ebase = wid * E_PER_W

    @pl.loop(0, ITERS)
    def _(i):
        base = ebase + i * C
        pltpu.sync_copy(src_hbm.at[pl.ds(base, C)], src_v)
        pltpu.sync_copy(dst_hbm.at[pl.ds(base, C)], dst_v)
        pltpu.async_copy(y_hbm.at[src_v], rows_v, sem).wait()
        pltpu.sync_copy(rows_v, acc.at[dst_v], add=True)

    plsc.subcore_barrier()
    pltpu.sync_copy(acc.at[pl.ds(rbase, RPS)], out_hbm.at[cid, pl.ds(rbase, RPS)])


def _tc_scalings(d0, d1, x_pad):
    def body(d0_ref, d1_ref, x_ref, y_ref, dis_ref, inv_ref):
        deg = d0_ref[:, 0:1] + d1_ref[:, 0:1] + 1.0
        dis = lax.rsqrt(deg)
        dis_ref[...] = dis
        inv_ref[...] = 1.0 / deg
        y_ref[...] = x_ref[...] * dis

    return pl.pallas_call(
        body,
        out_shape=(
            jax.ShapeDtypeStruct((N_PAD, D), _f32),
            jax.ShapeDtypeStruct((N_PAD, 1), _f32),
            jax.ShapeDtypeStruct((N_PAD, 1), _f32),
        ),
    )(d0, d1, x_pad)


def _tc_mid(q0, q1, y1, inv):
    def body(q0_ref, q1_ref, y1_ref, inv_ref, y2_ref):
        z = q0_ref[...] + q1_ref[...] + y1_ref[...]
        y2_ref[...] = z * inv_ref[...]

    return pl.pallas_call(
        body, out_shape=jax.ShapeDtypeStruct((N_PAD, D), _f32),
    )(q0, q1, y1, inv)


def _tc_final(r0, r1, y2, dis, W, b2):
    def body(r0_ref, r1_ref, y2_ref, dis_ref, w_ref, b_ref, out_ref):
        h = (r0_ref[...] + r1_ref[...] + y2_ref[...]) * dis_ref[...]
        out_ref[...] = lax.dot_general(
            h, w_ref[...], (((1,), (1,)), ((), ())),
            preferred_element_type=_f32,
        ) + b_ref[...]

    return pl.pallas_call(
        body, out_shape=jax.ShapeDtypeStruct((N_PAD, D), _f32),
    )(r0, r1, y2, dis, W, b2)


def kernel(edge_index, x, W, b):
    src = edge_index[0].astype(jnp.int32)
    dst = edge_index[1].astype(jnp.int32)
    pad = jnp.full((E_PAD - E,), N, jnp.int32)
    src_p = jnp.concatenate([src, pad])
    dst_p = jnp.concatenate([dst, pad])
    x_pad = jnp.pad(x, ((0, N_PAD - N), (0, 0)))
    zeros_d = jnp.zeros((N_PAD, D), _f32)

    dparts = _deg_kernel(dst_p, zeros_d, jnp.ones((C, D), _f32))
    y1, dis, inv = _tc_scalings(dparts[0], dparts[1], x_pad)
    qparts = _hop_kernel(y1, src_p, dst_p, zeros_d)
    y2 = _tc_mid(qparts[0], qparts[1], y1, inv)
    rparts = _hop_kernel(y2, src_p, dst_p, zeros_d)
    out = _tc_final(rparts[0], rparts[1], y2, dis, W, b.reshape(1, D))
    return out[:N]
